# baseline (speedup 1.0000x reference)
import jax
import jax.numpy as jnp
from jax import lax
from jax.experimental import pallas as pl
from jax.experimental.pallas import tpu as pltpu
from jax.experimental.pallas import tpu_sc as plsc

_EMB = 64
_NC = 2
_NS = 16
_NW = _NC * _NS

_BB = 128
_L = 16
_Q = 4


def _gather_body(idx_hbm, table_hbm, out_hbm,
                 idx0, idx1, rows0, rows1, st0, st1,
                 si0, si1, sg0, sg1, so0, so1):
  n_s1, n_b1, n_s2, _ = idx_hbm.shape
  seq = n_s1 * n_s2
  n_units = seq * n_b1
  per_w = n_units // _NW
  n_ss = per_w // _Q
  wid = lax.axis_index("s") * _NC + lax.axis_index("c")
  base = wid * per_w

  idxs = (idx0, idx1)
  rows = (rows0, rows1)
  stg = (st0, st1)
  sem_i = (si0, si1)
  sem_g = (sg0, sg1)
  sem_o = (so0, so1)

  iota = lax.iota(jnp.int32, _L)
  row_ids = [iota + _L * k for k in range(_BB // _L)]

  def unit_coords(u):
    s = u // n_b1
    b1 = u % n_b1
    return s // n_s2, s % n_s2, b1, s

  def load_idx(ss, s):
    u0 = base + ss * _Q
    s1, s2, b1, _ = unit_coords(u0)
    for q in range(_Q):
      pltpu.async_copy(
          idx_hbm.at[s1, b1 + q, s2], idxs[s].at[q], sem_i[s])

  def wait_idx(s):
    pltpu.make_async_copy(
        idx_hbm.at[0, pl.ds(0, _Q), 0], idxs[s], sem_i[s]).wait()

  def fire_gathers(s):
    for q in range(_Q):
      pltpu.async_copy(
          table_hbm.at[idxs[s].at[q]],
          rows[s].at[pl.ds(_BB * q, _BB)], sem_g[s])

  def wait_gathers(s):
    pltpu.make_async_copy(
        table_hbm.at[pl.ds(0, _Q * _BB)], rows[s], sem_g[s]).wait()

  def fire_stores(u, ts):
    _, _, b1, sq = unit_coords(u)
    for e1 in range(_EMB // 8):
      pltpu.async_copy(
          stg[ts].at[pl.ds(8 * e1, 8)], out_hbm.at[sq, e1, b1], sem_o[ts])

  def wait_stores(ts):
    for e1 in range(_EMB // 8):
      pltpu.make_async_copy(
          stg[ts].at[pl.ds(8 * e1, 8)], out_hbm.at[0, e1, 0],
          sem_o[ts]).wait()

  def transpose(s, q, ts):
    rq = [r + _BB * q for r in row_ids]
    def body(d, carry):
      perm = lax.rem(iota + d, _L)
      for m in range(_EMB // _L):
        colv = perm + _L * m
        for k in range(_BB // _L):
          vals = plsc.load_gather(rows[s], [rq[k], colv])
          plsc.store_scatter(stg[ts], [colv, row_ids[k]], vals)
      return carry
    lax.fori_loop(0, _L, body, 0)

  def superstep(ss, s):
    wait_gathers(s)

    @pl.when(ss + 2 < n_ss)
    def _():
      load_idx(ss + 2, s)

    @pl.when(ss + 1 < n_ss)
    def _():
      wait_idx(s ^ 1)
      fire_gathers(s ^ 1)

    for q in range(_Q):
      u = base + ss * _Q + q
      gu = ss * _Q + q
      ts = q % 2

      @pl.when(gu >= 2)
      def _():
        wait_stores(ts)

      transpose(s, q, ts)
      fire_stores(u, ts)

  load_idx(0, 0)
  load_idx(1, 1)
  wait_idx(0)
  fire_gathers(0)

  def pair(k, carry):
    superstep(2 * k, 0)
    superstep(2 * k + 1, 1)
    return carry

  lax.fori_loop(0, n_ss // 2, pair, 0)

  wait_stores(0)
  wait_stores(1)


import functools


@functools.partial(jax.jit, static_argnums=(2, 3))
def _embed_lookup(idx4d, table, b, s):
  run = pl.kernel(
      _gather_body,
      out_type=jax.ShapeDtypeStruct(
          (s, _EMB // 8, b // _BB, 8, _BB), jnp.float32),
      mesh=plsc.VectorSubcoreMesh(
          core_axis_name="c", subcore_axis_name="s",
          num_cores=_NC, num_subcores=_NS,
      ),
      scratch_types=(
          [pltpu.VMEM((_Q, _BB), jnp.int32)] * 2
          + [pltpu.VMEM((_Q * _BB, _EMB), jnp.float32)] * 2
          + [pltpu.VMEM((_EMB, _BB), jnp.float32)] * 2
          + [pltpu.SemaphoreType.DMA] * 6
      ),
      compiler_params=pltpu.CompilerParams(
          use_tc_tiling_on_sc=False, needs_layout_passes=False),
  )
  return run(idx4d, table)


def kernel(input_tensor, token_table):
  b, s = input_tensor.shape
  idx = input_tensor.astype(jnp.int32)
  idx4d = idx.reshape(b // _BB, _BB, s // 8, 8).transpose(2, 0, 3, 1)
  out5d = _embed_lookup(idx4d, token_table, b, s)
  return out5d.transpose(2, 4, 0, 1, 3).reshape(b, s, _EMB)


# speedup vs baseline: 1.0102x; 1.0102x over previous
import jax
import jax.numpy as jnp
from jax import lax
from jax.experimental import pallas as pl
from jax.experimental.pallas import tpu as pltpu
from jax.experimental.pallas import tpu_sc as plsc

_EMB = 64
_NC = 2
_NS = 16
_NW = _NC * _NS

_BB = 128
_L = 16
_Q = 4


def _gather_body(idx_hbm, table_hbm, out_hbm,
                 idx0, idx1, rows0, rows1, st0, st1,
                 si0, si1, sg0, sg1, so0, so1):
  n_s1, n_b1, n_s2, _ = idx_hbm.shape
  seq = n_s1 * n_s2
  n_units = seq * n_b1
  per_w = n_units // _NW
  n_ss = per_w // _Q
  wid = lax.axis_index("s") * _NC + lax.axis_index("c")
  base = wid * per_w

  idxs = (idx0, idx1)
  rows = (rows0, rows1)
  stg = (st0, st1)
  sem_i = (si0, si1)
  sem_g = (sg0, sg1)
  sem_o = (so0, so1)

  iota = lax.iota(jnp.int32, _L)
  row_ids = [iota + _L * k for k in range(_BB // _L)]

  def unit_coords(u):
    s = u // n_b1
    b1 = u % n_b1
    return s // n_s2, s % n_s2, b1, s

  def load_idx(ss, s):
    u0 = base + ss * _Q
    s1, s2, b1, _ = unit_coords(u0)
    pltpu.async_copy(
        idx_hbm.at[s1, pl.ds(b1, _Q), s2], idxs[s], sem_i[s])

  def wait_idx(s):
    pltpu.make_async_copy(
        idx_hbm.at[0, pl.ds(0, _Q), 0], idxs[s], sem_i[s]).wait()

  def fire_gathers(s):
    for q in range(_Q):
      pltpu.async_copy(
          table_hbm.at[idxs[s].at[q]],
          rows[s].at[pl.ds(_BB * q, _BB)], sem_g[s])

  def wait_gathers(s):
    pltpu.make_async_copy(
        table_hbm.at[pl.ds(0, _Q * _BB)], rows[s], sem_g[s]).wait()

  def fire_stores(u, ts):
    _, _, b1, sq = unit_coords(u)
    pltpu.async_copy(stg[ts], out_hbm.at[sq, :, b1], sem_o[ts])

  def wait_stores(ts):
    pltpu.make_async_copy(stg[ts], out_hbm.at[0, :, 0], sem_o[ts]).wait()

  def transpose(s, q, ts):
    rq = [r + _BB * q for r in row_ids]
    def body(d, carry):
      perm = lax.rem(iota + d, _L)
      for m in range(_EMB // _L):
        colv = perm + _L * m
        e1v = lax.shift_right_logical(colv, 3)
        e2v = lax.bitwise_and(colv, 7)
        for k in range(_BB // _L):
          vals = plsc.load_gather(rows[s], [rq[k], colv])
          plsc.store_scatter(stg[ts], [e1v, e2v, row_ids[k]], vals)
      return carry
    lax.fori_loop(0, _L, body, 0)

  def superstep(ss, s):
    wait_gathers(s)

    @pl.when(ss + 2 < n_ss)
    def _():
      load_idx(ss + 2, s)

    @pl.when(ss + 1 < n_ss)
    def _():
      wait_idx(s ^ 1)
      fire_gathers(s ^ 1)

    for q in range(_Q):
      u = base + ss * _Q + q
      gu = ss * _Q + q
      ts = q % 2

      @pl.when(gu >= 2)
      def _():
        wait_stores(ts)

      transpose(s, q, ts)
      fire_stores(u, ts)

  load_idx(0, 0)
  load_idx(1, 1)
  wait_idx(0)
  fire_gathers(0)

  def pair(k, carry):
    superstep(2 * k, 0)
    superstep(2 * k + 1, 1)
    return carry

  lax.fori_loop(0, n_ss // 2, pair, 0)

  wait_stores(0)
  wait_stores(1)


import functools


@functools.partial(jax.jit, static_argnums=(2, 3))
def _embed_lookup(idx4d, table, b, s):
  run = pl.kernel(
      _gather_body,
      out_type=jax.ShapeDtypeStruct(
          (s, _EMB // 8, b // _BB, 8, _BB), jnp.float32),
      mesh=plsc.VectorSubcoreMesh(
          core_axis_name="c", subcore_axis_name="s",
          num_cores=_NC, num_subcores=_NS,
      ),
      scratch_types=(
          [pltpu.VMEM((_Q, _BB), jnp.int32)] * 2
          + [pltpu.VMEM((_Q * _BB, _EMB), jnp.float32)] * 2
          + [pltpu.VMEM((_EMB // 8, 8, _BB), jnp.float32)] * 2
          + [pltpu.SemaphoreType.DMA] * 6
      ),
      compiler_params=pltpu.CompilerParams(
          use_tc_tiling_on_sc=False, needs_layout_passes=False),
  )
  return run(idx4d, table)


def kernel(input_tensor, token_table):
  b, s = input_tensor.shape
  idx = input_tensor.astype(jnp.int32)
  idx4d = idx.reshape(b // _BB, _BB, s // 8, 8).transpose(2, 0, 3, 1)
  out5d = _embed_lookup(idx4d, token_table, b, s)
  return out5d.transpose(2, 4, 0, 1, 3).reshape(b, s, _EMB)


# speedup vs baseline: 1.0204x; 1.0101x over previous
import jax
import jax.numpy as jnp
from jax import lax
from jax.experimental import pallas as pl
from jax.experimental.pallas import tpu as pltpu
from jax.experimental.pallas import tpu_sc as plsc

_EMB = 64
_NC = 2
_NS = 16
_NW = _NC * _NS

_BB = 128
_L = 16


def _gather_body(idx_hbm, table_hbm, out_hbm,
                 idx0, idx1, rows0, rows1, st0, st1,
                 si0, si1, sg0, sg1, so0, so1):
  n_s1, n_b1, n_s2, _ = idx_hbm.shape
  seq = n_s1 * n_s2
  n_units = seq * n_b1
  per_w = n_units // _NW
  wid = lax.axis_index("s") * _NC + lax.axis_index("c")
  base = wid * per_w

  idxs = (idx0, idx1)
  rows = (rows0, rows1)
  stg = (st0, st1)
  sem_i = (si0, si1)
  sem_g = (sg0, sg1)
  sem_o = (so0, so1)

  iota = lax.iota(jnp.int32, _L)
  row_ids = [iota + _L * k for k in range(_BB // _L)]

  def unit_coords(u):
    s = u // n_b1
    b1 = u % n_b1
    return s // n_s2, s % n_s2, b1, s

  def load_idx(u, s):
    s1, s2, b1, _ = unit_coords(u)
    pltpu.async_copy(idx_hbm.at[s1, b1, s2], idxs[s], sem_i[s])

  def wait_idx(s):
    pltpu.make_async_copy(idx_hbm.at[0, 0, 0], idxs[s], sem_i[s]).wait()

  def fire_gather(s):
    pltpu.async_copy(table_hbm.at[idxs[s]], rows[s], sem_g[s])

  def wait_gather(s):
    pltpu.make_async_copy(
        table_hbm.at[pl.ds(0, _BB)], rows[s], sem_g[s]).wait()

  def fire_stores(u, s):
    _, _, b1, sq = unit_coords(u)
    for e1 in range(_EMB // 8):
      pltpu.async_copy(
          stg[s].at[pl.ds(8 * e1, 8)], out_hbm.at[sq, e1, b1], sem_o[s])

  def wait_stores(s):
    for e1 in range(_EMB // 8):
      pltpu.make_async_copy(
          stg[s].at[pl.ds(8 * e1, 8)], out_hbm.at[0, e1, 0], sem_o[s]).wait()

  def transpose(s):
    def body(d, carry):
      perm = lax.rem(iota + d, _L)
      for m in range(_EMB // _L):
        colv = perm + _L * m
        for k in range(_BB // _L):
          vals = plsc.load_gather(rows[s], [row_ids[k], colv])
          plsc.store_scatter(stg[s], [colv, row_ids[k]], vals)
      return carry
    lax.fori_loop(0, _L, body, 0)

  def step(i, s):
    wait_gather(s)

    @pl.when(i + 2 < per_w)
    def _():
      load_idx(base + i + 2, s)

    @pl.when(i + 1 < per_w)
    def _():
      wait_idx(s ^ 1)
      fire_gather(s ^ 1)

    @pl.when(i >= 2)
    def _():
      wait_stores(s)

    transpose(s)
    fire_stores(base + i, s)

  load_idx(base, 0)
  load_idx(base + 1, 1)
  wait_idx(0)
  fire_gather(0)

  def pair(k, carry):
    step(2 * k, 0)
    step(2 * k + 1, 1)
    return carry

  lax.fori_loop(0, per_w // 2, pair, 0)

  wait_stores(0)
  wait_stores(1)


import functools


@functools.partial(jax.jit, static_argnums=(2, 3))
def _embed_lookup(idx4d, table, b, s):
  run = pl.kernel(
      _gather_body,
      out_type=jax.ShapeDtypeStruct(
          (s, _EMB // 8, b // _BB, 8, _BB), jnp.float32),
      mesh=plsc.VectorSubcoreMesh(
          core_axis_name="c", subcore_axis_name="s",
          num_cores=_NC, num_subcores=_NS,
      ),
      scratch_types=[
          pltpu.VMEM((_BB,), jnp.int32),
          pltpu.VMEM((_BB,), jnp.int32),
          pltpu.VMEM((_BB, _EMB), jnp.float32),
          pltpu.VMEM((_BB, _EMB), jnp.float32),
          pltpu.VMEM((_EMB, _BB), jnp.float32),
          pltpu.VMEM((_EMB, _BB), jnp.float32),
          pltpu.SemaphoreType.DMA,
          pltpu.SemaphoreType.DMA,
          pltpu.SemaphoreType.DMA,
          pltpu.SemaphoreType.DMA,
          pltpu.SemaphoreType.DMA,
          pltpu.SemaphoreType.DMA,
      ],
      compiler_params=pltpu.CompilerParams(
          use_tc_tiling_on_sc=False, needs_layout_passes=False),
  )
  return run(idx4d, table)


def kernel(input_tensor, token_table):
  b, s = input_tensor.shape
  idx = input_tensor.astype(jnp.int32)
  idx4d = idx.reshape(b // _BB, _BB, s // 8, 8).transpose(2, 0, 3, 1)
  out5d = _embed_lookup(idx4d, token_table, b, s)
  return out5d.transpose(2, 4, 0, 1, 3).reshape(b, s, _EMB)
